# baseline (speedup 1.0000x reference)
import functools

import jax
import jax.numpy as jnp
from jax import lax
from jax.experimental import pallas as pl
from jax.experimental.pallas import tpu as pltpu
from jax.experimental.pallas import tpu_sc as plsc

S = 16384
E = 64
NUM_ROWS = 1000

NC = 2
NS = 16
NW = NC * NS
TOK_PER_W = S // NW
CHUNK = 128
NCH = TOK_PER_W // CHUNK
OUT_W = 128
FIN_B = 2048
K_W = FIN_B // TOK_PER_W


def _project_body(entT_ref, relT_ref, w_ref, b_ref, tbl_ref):
    entT = entT_ref[...]
    relT = relT_ref[...]
    w = w_ref[...]
    dn = (((0,), (1,)), ((), ()))
    tbl_ref[0:NUM_ROWS, :] = lax.dot_general(
        entT, w[:, 0:E], dn, preferred_element_type=jnp.float32)
    tbl_ref[NUM_ROWS:2 * NUM_ROWS, :] = lax.dot_general(
        relT, w[:, E:2 * E], dn, preferred_element_type=jnp.float32) + b_ref[...]
    tbl_ref[2 * NUM_ROWS:3 * NUM_ROWS, :] = lax.dot_general(
        entT, w[:, 2 * E:3 * E], dn, preferred_element_type=jnp.float32)


def _project_tables(entity_emb, relation_emb, W_tok, b_tok):
    return pl.pallas_call(
        _project_body,
        out_shape=jax.ShapeDtypeStruct((3 * NUM_ROWS, E), jnp.float32),
    )(entity_emb.T, relation_emb.T, W_tok, b_tok.reshape(1, E))


def _sc_body(msT_hbm, tbl_hbm, out_hbm, ibuf, gh, gr, gt, semg, semw):
    wid = lax.axis_index("s") * NC + lax.axis_index("c")
    base = pl.multiple_of(wid * TOK_PER_W, TOK_PER_W)
    row0 = pl.multiple_of(
        (wid // (2 * K_W)) * FIN_B + (wid % K_W) * TOK_PER_W, TOK_PER_W)
    col0 = ((wid % (2 * K_W)) // K_W) * E

    pltpu.sync_copy(msT_hbm.at[:, pl.ds(base, TOK_PER_W)], ibuf)

    def _offsets(g, carry):
        sl = pl.ds(g * 16, 16)
        ibuf[1, sl] = ibuf[1, sl] + NUM_ROWS
        ibuf[2, sl] = ibuf[2, sl] + 2 * NUM_ROWS
        return carry

    lax.fori_loop(0, TOK_PER_W // 16, _offsets, 0)

    def _gathers(c):
        sl = pl.ds(c * CHUNK, CHUNK)
        return (pltpu.make_async_copy(tbl_hbm.at[ibuf.at[0, sl]], gh.at[c], semg),
                pltpu.make_async_copy(tbl_hbm.at[ibuf.at[1, sl]], gr.at[c], semg),
                pltpu.make_async_copy(tbl_hbm.at[ibuf.at[2, sl]], gt.at[c], semg))

    def _writeback(c):
        return pltpu.make_async_copy(
            gh.at[c],
            out_hbm.at[pl.ds(row0 + c * CHUNK, CHUNK), pl.ds(col0, E)],
            semw)

    def _start(c, carry):
        for d in _gathers(c):
            d.start()
        return carry

    lax.fori_loop(0, NCH, _start, 0)

    def _chunk(c, carry):
        for d in _gathers(c):
            d.wait()

        def body(i, c2):
            for j in range(E // 16):
                sl = pl.ds(j * 16, 16)
                plsc.addupdate(gh.at[c, i, sl], gr[c, i, sl] + gt[c, i, sl])
            return c2

        lax.fori_loop(0, CHUNK, body, 0)
        _writeback(c).start()
        return carry

    lax.fori_loop(0, NCH, _chunk, 0)

    def _drain(c, carry):
        _writeback(c).wait()
        return carry

    lax.fori_loop(0, NCH, _drain, 0)


_sc_gather = functools.partial(
    pl.kernel,
    out_type=jax.ShapeDtypeStruct((S * E // OUT_W, OUT_W), jnp.float32),
    mesh=plsc.VectorSubcoreMesh(core_axis_name="c", subcore_axis_name="s"),
    scratch_types=[
        pltpu.VMEM((3, TOK_PER_W), jnp.int32),
        pltpu.VMEM((NCH, CHUNK, E), jnp.float32),
        pltpu.VMEM((NCH, CHUNK, E), jnp.float32),
        pltpu.VMEM((NCH, CHUNK, E), jnp.float32),
        pltpu.SemaphoreType.DMA,
        pltpu.SemaphoreType.DMA,
    ],
    compiler_params=pltpu.CompilerParams(use_tc_tiling_on_sc=False,
                                         needs_layout_passes=False),
)(_sc_body)


def _finish_body(in_ref, out_ref):
    t = in_ref[...].T
    out_ref[:, 0:FIN_B] = t[0:E, :]
    out_ref[:, FIN_B:2 * FIN_B] = t[E:OUT_W, :]


def _finish(packed):
    grid = (S * E // OUT_W) // FIN_B
    return pl.pallas_call(
        _finish_body,
        grid=(grid,),
        in_specs=[pl.BlockSpec((FIN_B, OUT_W), lambda g: (g, 0))],
        out_specs=pl.BlockSpec((E, 2 * FIN_B), lambda g: (0, g)),
        out_shape=jax.ShapeDtypeStruct((E, S), jnp.float32),
    )(packed)


def kernel(memory_state, entity_emb, relation_emb, W_tok, b_tok):
    tbl = _project_tables(entity_emb, relation_emb, W_tok, b_tok)
    packed = _sc_gather(memory_state.T, tbl)
    return _finish(packed).T


# speedup vs baseline: 1.1090x; 1.1090x over previous
import functools

import jax
import jax.numpy as jnp
from jax import lax
from jax.experimental import pallas as pl
from jax.experimental.pallas import tpu as pltpu
from jax.experimental.pallas import tpu_sc as plsc

S = 16384
E = 64
NUM_ROWS = 1000

NC = 2
NS = 16
NW = NC * NS
TOK_PER_W = S // NW
CHUNK = 128
NCH = TOK_PER_W // CHUNK
OUT_W = 128
FIN_B = 2048
K_W = FIN_B // TOK_PER_W


def _project_body(entT_ref, relT_ref, w_ref, b_ref, tbl_ref):
    entT = entT_ref[...]
    relT = relT_ref[...]
    w = w_ref[...]
    dn = (((0,), (1,)), ((), ()))
    tbl_ref[0:NUM_ROWS, :] = lax.dot_general(
        entT, w[:, 0:E], dn, preferred_element_type=jnp.float32)
    tbl_ref[NUM_ROWS:2 * NUM_ROWS, :] = lax.dot_general(
        relT, w[:, E:2 * E], dn, preferred_element_type=jnp.float32) + b_ref[...]
    tbl_ref[2 * NUM_ROWS:3 * NUM_ROWS, :] = lax.dot_general(
        entT, w[:, 2 * E:3 * E], dn, preferred_element_type=jnp.float32)


def _project_tables(entity_emb, relation_emb, W_tok, b_tok):
    return pl.pallas_call(
        _project_body,
        out_shape=jax.ShapeDtypeStruct((3 * NUM_ROWS, E), jnp.float32),
    )(entity_emb.T, relation_emb.T, W_tok, b_tok.reshape(1, E))


def _sc_body(msT_hbm, tbl_hbm, out_hbm, ibuf, gh, gr, gt, semg, semw):
    wid = lax.axis_index("s") * NC + lax.axis_index("c")
    base = pl.multiple_of(wid * TOK_PER_W, TOK_PER_W)
    row0 = pl.multiple_of(
        (wid // (2 * K_W)) * FIN_B + (wid % K_W) * TOK_PER_W, TOK_PER_W)
    col0 = ((wid % (2 * K_W)) // K_W) * E

    pltpu.sync_copy(msT_hbm.at[:, pl.ds(base, TOK_PER_W)], ibuf)
    for g in range(TOK_PER_W // 16):
        sl = pl.ds(g * 16, 16)
        ibuf[1, sl] = ibuf[1, sl] + NUM_ROWS
        ibuf[2, sl] = ibuf[2, sl] + 2 * NUM_ROWS

    def start_gathers(c):
        sl = pl.ds(c * CHUNK, CHUNK)
        return (pltpu.async_copy(tbl_hbm.at[ibuf.at[0, sl]], gh.at[c], semg),
                pltpu.async_copy(tbl_hbm.at[ibuf.at[1, sl]], gr.at[c], semg),
                pltpu.async_copy(tbl_hbm.at[ibuf.at[2, sl]], gt.at[c], semg))

    def compute(c):
        def body(i, carry):
            for j in range(E // 16):
                sl = pl.ds(j * 16, 16)
                plsc.addupdate(gh.at[c, i, sl], gr[c, i, sl] + gt[c, i, sl])
            return carry

        lax.fori_loop(0, CHUNK, body, 0)

    pending = [start_gathers(c) for c in range(NCH)]
    wbs = []
    for c in range(NCH):
        for d in pending[c]:
            d.wait()
        compute(c)
        wbs.append(pltpu.async_copy(
            gh.at[c],
            out_hbm.at[pl.ds(row0 + c * CHUNK, CHUNK), pl.ds(col0, E)],
            semw))
    for w in wbs:
        w.wait()


_sc_gather = functools.partial(
    pl.kernel,
    out_type=jax.ShapeDtypeStruct((S * E // OUT_W, OUT_W), jnp.float32),
    mesh=plsc.VectorSubcoreMesh(core_axis_name="c", subcore_axis_name="s"),
    scratch_types=[
        pltpu.VMEM((3, TOK_PER_W), jnp.int32),
        pltpu.VMEM((NCH, CHUNK, E), jnp.float32),
        pltpu.VMEM((NCH, CHUNK, E), jnp.float32),
        pltpu.VMEM((NCH, CHUNK, E), jnp.float32),
        pltpu.SemaphoreType.DMA,
        pltpu.SemaphoreType.DMA,
    ],
    compiler_params=pltpu.CompilerParams(use_tc_tiling_on_sc=False,
                                         needs_layout_passes=False,
                                         skip_device_barrier=True),
)(_sc_body)


def _finish_body(in_ref, out_ref):
    t = in_ref[...].T
    out_ref[:, 0:FIN_B] = t[0:E, :]
    out_ref[:, FIN_B:2 * FIN_B] = t[E:OUT_W, :]


def _finish(packed):
    grid = (S * E // OUT_W) // FIN_B
    return pl.pallas_call(
        _finish_body,
        grid=(grid,),
        in_specs=[pl.BlockSpec((FIN_B, OUT_W), lambda g: (g, 0))],
        out_specs=pl.BlockSpec((E, 2 * FIN_B), lambda g: (0, g)),
        out_shape=jax.ShapeDtypeStruct((E, S), jnp.float32),
    )(packed)


def kernel(memory_state, entity_emb, relation_emb, W_tok, b_tok):
    tbl = _project_tables(entity_emb, relation_emb, W_tok, b_tok)
    packed = _sc_gather(memory_state.T, tbl)
    return _finish(packed).T


# speedup vs baseline: 1.1862x; 1.0695x over previous
import functools

import jax
import jax.numpy as jnp
from jax import lax
from jax.experimental import pallas as pl
from jax.experimental.pallas import tpu as pltpu
from jax.experimental.pallas import tpu_sc as plsc

S = 16384
E = 64
NUM_ROWS = 1000

NC = 2
NS = 16
NW = NC * NS
TOK_PER_W = S // NW
CHUNK = 128
NCH = TOK_PER_W // CHUNK
OUT_W = 128
FIN_B = 2048
K_W = FIN_B // TOK_PER_W


TBL_PAD = 3072


def _project_body(entT_ref, relT_ref, w_ref, b_ref, tbl_ref):
    entT = entT_ref[...]
    relT = relT_ref[...]
    w = w_ref[...]
    dn = (((0,), (1,)), ((), ()))
    tbl_ref[0:NUM_ROWS, :] = lax.dot_general(
        entT, w[:, 0:E], dn, preferred_element_type=jnp.float32)
    tbl_ref[NUM_ROWS:2 * NUM_ROWS, :] = lax.dot_general(
        relT, w[:, E:2 * E], dn, preferred_element_type=jnp.float32) + b_ref[...]
    tbl_ref[2 * NUM_ROWS:3 * NUM_ROWS, :] = lax.dot_general(
        entT, w[:, 2 * E:3 * E], dn, preferred_element_type=jnp.float32)
    tbl_ref[3 * NUM_ROWS:TBL_PAD, :] = jnp.zeros(
        (TBL_PAD - 3 * NUM_ROWS, E), jnp.float32)


def _project_tables(entity_emb, relation_emb, W_tok, b_tok):
    return pl.pallas_call(
        _project_body,
        out_shape=jax.ShapeDtypeStruct((TBL_PAD, E), jnp.float32),
    )(entity_emb.T, relation_emb.T, W_tok, b_tok.reshape(1, E))


def _sc_body(msT_hbm, tbl_hbm, out_hbm, ibuf, gh, gr, gt, tbl_sp,
             semg, semw):
    sid = lax.axis_index("s")
    wid = sid * NC + lax.axis_index("c")

    trow = pl.multiple_of(sid * (TBL_PAD // NS), TBL_PAD // NS)
    pltpu.sync_copy(tbl_hbm.at[pl.ds(trow, TBL_PAD // NS), :],
                    tbl_sp.at[pl.ds(trow, TBL_PAD // NS), :])
    base = pl.multiple_of(wid * TOK_PER_W, TOK_PER_W)
    row0 = pl.multiple_of(
        (wid // (2 * K_W)) * FIN_B + (wid % K_W) * TOK_PER_W, TOK_PER_W)
    col0 = ((wid % (2 * K_W)) // K_W) * E

    pltpu.sync_copy(msT_hbm.at[:, pl.ds(base, TOK_PER_W)], ibuf)
    for g in range(TOK_PER_W // 16):
        sl = pl.ds(g * 16, 16)
        ibuf[1, sl] = ibuf[1, sl] + NUM_ROWS
        ibuf[2, sl] = ibuf[2, sl] + 2 * NUM_ROWS

    plsc.subcore_barrier()

    def start_gathers(c):
        sl = pl.ds(c * CHUNK, CHUNK)
        return (pltpu.async_copy(tbl_sp.at[ibuf.at[0, sl]], gh.at[c], semg),
                pltpu.async_copy(tbl_sp.at[ibuf.at[1, sl]], gr.at[c], semg),
                pltpu.async_copy(tbl_sp.at[ibuf.at[2, sl]], gt.at[c], semg))

    def compute(c):
        def body(i, carry):
            for j in range(E // 16):
                sl = pl.ds(j * 16, 16)
                plsc.addupdate(gh.at[c, i, sl], gr[c, i, sl] + gt[c, i, sl])
            return carry

        lax.fori_loop(0, CHUNK, body, 0)

    pending = [start_gathers(c) for c in range(NCH)]
    wbs = []
    for c in range(NCH):
        for d in pending[c]:
            d.wait()
        compute(c)
        wbs.append(pltpu.async_copy(
            gh.at[c],
            out_hbm.at[pl.ds(row0 + c * CHUNK, CHUNK), pl.ds(col0, E)],
            semw))
    for w in wbs:
        w.wait()


_sc_gather = functools.partial(
    pl.kernel,
    out_type=jax.ShapeDtypeStruct((S * E // OUT_W, OUT_W), jnp.float32),
    mesh=plsc.VectorSubcoreMesh(core_axis_name="c", subcore_axis_name="s"),
    scratch_types=[
        pltpu.VMEM((3, TOK_PER_W), jnp.int32),
        pltpu.VMEM((NCH, CHUNK, E), jnp.float32),
        pltpu.VMEM((NCH, CHUNK, E), jnp.float32),
        pltpu.VMEM((NCH, CHUNK, E), jnp.float32),
        pltpu.VMEM_SHARED((TBL_PAD, E), jnp.float32),
        pltpu.SemaphoreType.DMA,
        pltpu.SemaphoreType.DMA,
    ],
    compiler_params=pltpu.CompilerParams(use_tc_tiling_on_sc=False,
                                         needs_layout_passes=False,
                                         skip_device_barrier=True),
)(_sc_body)


def _finish_body(in_ref, out_ref):
    t = in_ref[...].T
    out_ref[:, 0:FIN_B] = t[0:E, :]
    out_ref[:, FIN_B:2 * FIN_B] = t[E:OUT_W, :]


def _finish(packed):
    grid = (S * E // OUT_W) // FIN_B
    return pl.pallas_call(
        _finish_body,
        grid=(grid,),
        in_specs=[pl.BlockSpec((FIN_B, OUT_W), lambda g: (g, 0))],
        out_specs=pl.BlockSpec((E, 2 * FIN_B), lambda g: (0, g)),
        out_shape=jax.ShapeDtypeStruct((E, S), jnp.float32),
    )(packed)


def kernel(memory_state, entity_emb, relation_emb, W_tok, b_tok):
    tbl = _project_tables(entity_emb, relation_emb, W_tok, b_tok)
    packed = _sc_gather(memory_state.T, tbl)
    return _finish(packed).T
